# Initial kernel scaffold; baseline (speedup 1.0000x reference)
import jax
import jax.numpy as jnp
from jax.experimental import pallas as pl


def kernel(x, edge_index, layers, dense_w, dense_b):
    raise NotImplementedError("write your pallas kernel here")


# speedup vs baseline: 3.1680x; 3.1680x over previous
import functools

import jax
import jax.numpy as jnp
from jax import lax
from jax.experimental import pallas as pl
from jax.experimental.pallas import tpu as pltpu
from jax.experimental.pallas import tpu_sc as plsc

_N = 10000
_E = 320000
_CB = 128
_NCB = 4
_ROWS = 1000
_NRB = _N // _ROWS
_NSC = 2
_NTILE = 16
_TPB = 624
_ZCH = 208
_TAIL = _N - _NTILE * _TPB
_ECH = 128
_EPAD = ((_E + _NTILE * _ECH * _NSC - 1) // (_NTILE * _ECH * _NSC)) * (_NTILE * _ECH * _NSC)
_PAD = _EPAD - _E

@functools.cache
def _sc_mesh():
    return plsc.VectorSubcoreMesh(
        core_axis_name="c", subcore_axis_name="s",
        num_cores=_NSC, num_subcores=_NTILE)


def _deg_body(row_hbm, out_hbm, shared, onesb, rbuf, zbuf, sem):
    c = lax.axis_index("c")
    s = lax.axis_index("s")
    zeros16 = jnp.zeros((16,), jnp.float32)
    ones16 = jnp.ones((16,), jnp.float32)

    def fill(i, _):
        for j in range(_CB // 16):
            onesb[i, pl.ds(j * 16, 16)] = ones16
        return 0
    lax.fori_loop(0, _ECH, fill, 0)

    def zfill(i, _):
        for j in range(_CB // 16):
            zbuf[i, pl.ds(j * 16, 16)] = zeros16
        return 0
    lax.fori_loop(0, _ZCH, zfill, 0)
    for k in range(3):
        pltpu.sync_copy(zbuf, shared.at[pl.ds(s * _TPB + k * _ZCH, _ZCH)])

    @pl.when(s == _NTILE - 1)
    def _():
        pltpu.sync_copy(zbuf.at[pl.ds(0, _TAIL)],
                        shared.at[pl.ds(_NTILE * _TPB, _TAIL)])
    plsc.subcore_barrier()

    ept = _EPAD // (_NSC * _NTILE)
    base0 = (c * _NTILE + s) * ept

    def chunk(k, _):
        eb = base0 + k * _ECH
        pltpu.sync_copy(row_hbm.at[pl.ds(eb, _ECH)], rbuf)
        pltpu.sync_copy(onesb, shared.at[rbuf], add=True)
        return 0
    lax.fori_loop(0, ept // _ECH, chunk, 0)
    plsc.subcore_barrier()
    pltpu.sync_copy(shared.at[pl.ds(s * _TPB, _TPB)],
                    out_hbm.at[c, pl.ds(s * _TPB, _TPB)])

    @pl.when(s == _NTILE - 1)
    def _():
        pltpu.sync_copy(shared.at[pl.ds(_NTILE * _TPB, _TAIL)],
                        out_hbm.at[c, pl.ds(_NTILE * _TPB, _TAIL)])


@functools.cache
def _deg_kernel():
    return pl.kernel(
        _deg_body,
        out_type=jax.ShapeDtypeStruct((_NSC, _N, _CB), jnp.float32),
        mesh=_sc_mesh(),
        scratch_types=[
            pltpu.VMEM_SHARED((_N + 8, _CB), jnp.float32),
            pltpu.VMEM((_ECH, _CB), jnp.float32),
            pltpu.VMEM((_ECH,), jnp.int32),
            pltpu.VMEM((_ZCH, _CB), jnp.float32),
            pltpu.SemaphoreType.DMA,
        ],
    )


def _deg_call(rowp):
    return _deg_kernel()(rowp)


def _agg_body(y_hbm, col_hbm, row_hbm, out_hbm,
              shared, rows_v, cbuf, gbuf, rbuf, zbuf, sem):
    c = lax.axis_index("c")
    s = lax.axis_index("s")
    zeros16 = jnp.zeros((16,), jnp.float32)

    def zfill(i, _):
        for j in range(8):
            zbuf[i, pl.ds(j * 16, 16)] = zeros16
        return 0
    lax.fori_loop(0, _ZCH, zfill, 0)

    ept = _EPAD // _NTILE
    base0 = s * ept
    for b in range(2):
        blk = c * 2 + b
        off = blk * _N
        for k in range(3):
            pltpu.sync_copy(zbuf, shared.at[pl.ds(s * _TPB + k * _ZCH, _ZCH)])

        @pl.when(s == _NTILE - 1)
        def _():
            pltpu.sync_copy(zbuf.at[pl.ds(0, _TAIL)],
                            shared.at[pl.ds(_NTILE * _TPB, _TAIL)])
        plsc.subcore_barrier()

        def chunk(k, _):
            eb = base0 + k * _ECH
            pltpu.sync_copy(col_hbm.at[pl.ds(eb, _ECH)], cbuf)
            pltpu.sync_copy(row_hbm.at[pl.ds(eb, _ECH)], rbuf)
            for j in range(_ECH // 16):
                gbuf[pl.ds(j * 16, 16)] = cbuf[pl.ds(j * 16, 16)] + off
            pltpu.async_copy(y_hbm.at[gbuf], rows_v, sem).wait()
            pltpu.sync_copy(rows_v, shared.at[rbuf], add=True)
            return 0
        lax.fori_loop(0, ept // _ECH, chunk, 0)
        plsc.subcore_barrier()
        pltpu.sync_copy(shared.at[pl.ds(s * _TPB, _TPB)],
                        out_hbm.at[pl.ds(blk * _N + s * _TPB, _TPB)])

        @pl.when(s == _NTILE - 1)
        def _():
            pltpu.sync_copy(shared.at[pl.ds(_NTILE * _TPB, _TAIL)],
                            out_hbm.at[pl.ds(blk * _N + _NTILE * _TPB, _TAIL)])
        plsc.subcore_barrier()


@functools.cache
def _agg_kernel():
    return pl.kernel(
        _agg_body,
        out_type=jax.ShapeDtypeStruct((_NCB * _N, _CB), jnp.float32),
        mesh=_sc_mesh(),
        scratch_types=[
            pltpu.VMEM_SHARED((_N + 8, _CB), jnp.float32),
            pltpu.VMEM((_ECH, _CB), jnp.float32),
            pltpu.VMEM((_ECH,), jnp.int32),
            pltpu.VMEM((_ECH,), jnp.int32),
            pltpu.VMEM((_ECH,), jnp.int32),
            pltpu.VMEM((_ZCH, _CB), jnp.float32),
            pltpu.SemaphoreType.DMA,
        ],
    )


def _agg_call(y4, colp, rowp):
    return _agg_kernel()(y4, colp, rowp)


def _dinv_from(parts):
    deg = parts[0, :, 0] + parts[1, :, 0]
    return jnp.where(deg > 0, lax.rsqrt(deg), 0.0)


def _mm1_body(parts_ref, h_ref, w_ref, y_ref):
    dinv = _dinv_from(parts_ref[...])
    y_ref[...] = dinv[:, None] * jnp.dot(
        h_ref[...], w_ref[...], preferred_element_type=jnp.float32)


def _mm1(parts, h, w1):
    in_dim = h.shape[1]
    return pl.pallas_call(
        _mm1_body,
        grid=(_NRB, _NCB),
        in_specs=[
            pl.BlockSpec((_NSC, _ROWS, _CB), lambda i, j: (0, i, 0)),
            pl.BlockSpec((_ROWS, in_dim), lambda i, j: (i, 0)),
            pl.BlockSpec((in_dim, _CB), lambda i, j: (0, j)),
        ],
        out_specs=pl.BlockSpec((_ROWS, _CB), lambda i, j: (j * _NRB + i, 0)),
        out_shape=jax.ShapeDtypeStruct((_NCB * _N, _CB), jnp.float32),
    )(parts, h, w1)


def _mm2_body(parts_ref, raw_ref, h_ref, w_ref, b_ref, o_ref):
    dinv = _dinv_from(parts_ref[...])
    z = (dinv[:, None] * raw_ref[...]
         + jnp.dot(h_ref[...], w_ref[...], preferred_element_type=jnp.float32)
         + b_ref[...][None, :])
    o_ref[...] = jnp.maximum(z, 0.0)


def _mm2(parts, raw4, h, w2, b):
    in_dim = h.shape[1]
    return pl.pallas_call(
        _mm2_body,
        grid=(_NRB, _NCB),
        in_specs=[
            pl.BlockSpec((_NSC, _ROWS, _CB), lambda i, j: (0, i, 0)),
            pl.BlockSpec((_ROWS, _CB), lambda i, j: (j * _NRB + i, 0)),
            pl.BlockSpec((_ROWS, in_dim), lambda i, j: (i, 0)),
            pl.BlockSpec((in_dim, _CB), lambda i, j: (0, j)),
            pl.BlockSpec((_CB,), lambda i, j: (j,)),
        ],
        out_specs=pl.BlockSpec((_ROWS, _CB), lambda i, j: (i, j)),
        out_shape=jax.ShapeDtypeStruct((_N, _NCB * _CB), jnp.float32),
    )(parts, raw4, h, w2, b)


def _dense_body(h_ref, w_ref, b_ref, o_ref):
    o_ref[...] = jnp.dot(
        h_ref[...], w_ref[...],
        preferred_element_type=jnp.float32) + b_ref[...][None, :]


def _dense(h, dw, db):
    k, o = dw.shape
    return pl.pallas_call(
        _dense_body,
        grid=(_NRB,),
        in_specs=[
            pl.BlockSpec((_ROWS, k), lambda i: (i, 0)),
            pl.BlockSpec((k, o), lambda i: (0, 0)),
            pl.BlockSpec((o,), lambda i: (0,)),
        ],
        out_specs=pl.BlockSpec((_ROWS, o), lambda i: (i, 0)),
        out_shape=jax.ShapeDtypeStruct((_N, o), jnp.float32),
    )(h, dw, db)


def kernel(x, edge_index, layers, dense_w, dense_b):
    row = edge_index[0]
    col = edge_index[1]
    rowp = jnp.concatenate([row, jnp.full((_PAD,), _N, jnp.int32)])
    colp = jnp.concatenate([col, jnp.zeros((_PAD,), jnp.int32)])
    parts = _deg_call(rowp)
    h = x
    for (w1, w2, b) in layers:
        y4 = _mm1(parts, h, w1)
        raw4 = _agg_call(y4, colp, rowp)
        h = _mm2(parts, raw4, h, w2, b)
    return _dense(h, dense_w, dense_b)


# speedup vs baseline: 4.3714x; 1.3799x over previous
import functools

import jax
import jax.numpy as jnp
from jax import lax
from jax.experimental import pallas as pl
from jax.experimental.pallas import tpu as pltpu
from jax.experimental.pallas import tpu_sc as plsc

_N = 10000
_E = 320000
_CB = 128
_NCB = 4
_ROWS = 1000
_NRB = _N // _ROWS
_NSC = 2
_NTILE = 16
_TPB = 624
_ZCH = 104
_NZC = _TPB // _ZCH
_TAIL = _N - _NTILE * _TPB
_ECH = 128
_EPAD = ((_E + _NTILE * _ECH * _NSC - 1) // (_NTILE * _ECH * _NSC)) * (_NTILE * _ECH * _NSC)
_PAD = _EPAD - _E

@functools.cache
def _sc_mesh():
    return plsc.VectorSubcoreMesh(
        core_axis_name="c", subcore_axis_name="s",
        num_cores=_NSC, num_subcores=_NTILE)


def _deg_body(row_hbm, out_hbm, shared, onesb, rbuf, zbuf, sem):
    c = lax.axis_index("c")
    s = lax.axis_index("s")
    zeros16 = jnp.zeros((16,), jnp.float32)
    ones16 = jnp.ones((16,), jnp.float32)

    def fill(i, _):
        for j in range(_CB // 16):
            onesb[i, pl.ds(j * 16, 16)] = ones16
        return 0
    lax.fori_loop(0, _ECH, fill, 0)

    def zfill(i, _):
        for j in range(_CB // 16):
            zbuf[i, pl.ds(j * 16, 16)] = zeros16
        return 0
    lax.fori_loop(0, _ZCH, zfill, 0)
    for k in range(_NZC):
        pltpu.sync_copy(zbuf, shared.at[pl.ds(s * _TPB + k * _ZCH, _ZCH)])

    @pl.when(s == _NTILE - 1)
    def _():
        pltpu.sync_copy(zbuf.at[pl.ds(0, _TAIL)],
                        shared.at[pl.ds(_NTILE * _TPB, _TAIL)])
    plsc.subcore_barrier()

    ept = _EPAD // (_NSC * _NTILE)
    base0 = (c * _NTILE + s) * ept

    def chunk(k, _):
        eb = base0 + k * _ECH
        pltpu.sync_copy(row_hbm.at[pl.ds(eb, _ECH)], rbuf)
        pltpu.sync_copy(onesb, shared.at[rbuf], add=True)
        return 0
    lax.fori_loop(0, ept // _ECH, chunk, 0)
    plsc.subcore_barrier()
    pltpu.sync_copy(shared.at[pl.ds(s * _TPB, _TPB)],
                    out_hbm.at[c, pl.ds(s * _TPB, _TPB)])

    @pl.when(s == _NTILE - 1)
    def _():
        pltpu.sync_copy(shared.at[pl.ds(_NTILE * _TPB, _TAIL)],
                        out_hbm.at[c, pl.ds(_NTILE * _TPB, _TAIL)])


@functools.cache
def _deg_kernel():
    return pl.kernel(
        _deg_body,
        out_type=jax.ShapeDtypeStruct((_NSC, _N, _CB), jnp.float32),
        mesh=_sc_mesh(),
        scratch_types=[
            pltpu.VMEM_SHARED((_N + 8, _CB), jnp.float32),
            pltpu.VMEM((_ECH, _CB), jnp.float32),
            pltpu.VMEM((_ECH,), jnp.int32),
            pltpu.VMEM((_ZCH, _CB), jnp.float32),
            pltpu.SemaphoreType.DMA,
        ],
    )


def _deg_call(rowp):
    return _deg_kernel()(rowp)


def _agg_body(y_hbm, col4_hbm, row_hbm, out_hbm,
              shared, rv0, rv1, cb0, cb1, rb0, rb1, zbuf, sg0, sg1):
    c = lax.axis_index("c")
    s = lax.axis_index("s")
    zeros16 = jnp.zeros((16,), jnp.float32)

    def zfill(i, _):
        for j in range(8):
            zbuf[i, pl.ds(j * 16, 16)] = zeros16
        return 0
    lax.fori_loop(0, _ZCH, zfill, 0)

    ept = _EPAD // _NTILE
    nch = ept // _ECH
    base0 = s * ept
    for b in range(2):
        blk = c * 2 + b
        for k in range(_NZC):
            pltpu.sync_copy(zbuf, shared.at[pl.ds(s * _TPB + k * _ZCH, _ZCH)])

        @pl.when(s == _NTILE - 1)
        def _():
            pltpu.sync_copy(zbuf.at[pl.ds(0, _TAIL)],
                            shared.at[pl.ds(_NTILE * _TPB, _TAIL)])
        plsc.subcore_barrier()

        def load_idx(k, cb, rb):
            eb = base0 + k * _ECH
            pltpu.sync_copy(col4_hbm.at[blk, pl.ds(eb, _ECH)], cb)
            pltpu.sync_copy(row_hbm.at[pl.ds(eb, _ECH)], rb)

        def start_g(cb, rv, sg):
            pltpu.async_copy(y_hbm.at[cb], rv, sg)

        def wait_g(cb, rv, sg):
            pltpu.make_async_copy(y_hbm.at[cb], rv, sg).wait()

        def scat(rv, rb):
            pltpu.sync_copy(rv, shared.at[rb], add=True)

        load_idx(0, cb0, rb0)
        start_g(cb0, rv0, sg0)
        load_idx(1, cb1, rb1)

        def step(m, _):
            k0 = 2 * m
            start_g(cb1, rv1, sg1)
            wait_g(cb0, rv0, sg0)
            scat(rv0, rb0)
            load_idx(k0 + 2, cb0, rb0)
            start_g(cb0, rv0, sg0)
            wait_g(cb1, rv1, sg1)
            scat(rv1, rb1)
            load_idx(k0 + 3, cb1, rb1)
            return 0
        lax.fori_loop(0, nch // 2 - 1, step, 0)
        start_g(cb1, rv1, sg1)
        wait_g(cb0, rv0, sg0)
        scat(rv0, rb0)
        wait_g(cb1, rv1, sg1)
        scat(rv1, rb1)
        plsc.subcore_barrier()
        pltpu.sync_copy(shared.at[pl.ds(s * _TPB, _TPB)],
                        out_hbm.at[pl.ds(blk * _N + s * _TPB, _TPB)])

        @pl.when(s == _NTILE - 1)
        def _():
            pltpu.sync_copy(shared.at[pl.ds(_NTILE * _TPB, _TAIL)],
                            out_hbm.at[pl.ds(blk * _N + _NTILE * _TPB, _TAIL)])
        plsc.subcore_barrier()


@functools.cache
def _agg_kernel():
    return pl.kernel(
        _agg_body,
        out_type=jax.ShapeDtypeStruct((_NCB * _N, _CB), jnp.float32),
        mesh=_sc_mesh(),
        scratch_types=[
            pltpu.VMEM_SHARED((_N + 8, _CB), jnp.float32),
            pltpu.VMEM((_ECH, _CB), jnp.float32),
            pltpu.VMEM((_ECH, _CB), jnp.float32),
            pltpu.VMEM((_ECH,), jnp.int32),
            pltpu.VMEM((_ECH,), jnp.int32),
            pltpu.VMEM((_ECH,), jnp.int32),
            pltpu.VMEM((_ECH,), jnp.int32),
            pltpu.VMEM((_ZCH, _CB), jnp.float32),
            pltpu.SemaphoreType.DMA,
            pltpu.SemaphoreType.DMA,
        ],
    )


def _agg_call(y4, col4, rowp):
    return _agg_kernel()(y4, col4, rowp)


def _dinv_from(parts):
    deg = parts[0, :, 0] + parts[1, :, 0]
    return jnp.where(deg > 0, lax.rsqrt(deg), 0.0)


def _mm1_body(parts_ref, h_ref, w_ref, y_ref):
    dinv = _dinv_from(parts_ref[...])
    y_ref[...] = dinv[:, None] * jnp.dot(
        h_ref[...], w_ref[...], preferred_element_type=jnp.float32)


def _mm1(parts, h, w1):
    in_dim = h.shape[1]
    return pl.pallas_call(
        _mm1_body,
        grid=(_NRB, _NCB),
        in_specs=[
            pl.BlockSpec((_NSC, _ROWS, _CB), lambda i, j: (0, i, 0)),
            pl.BlockSpec((_ROWS, in_dim), lambda i, j: (i, 0)),
            pl.BlockSpec((in_dim, _CB), lambda i, j: (0, j)),
        ],
        out_specs=pl.BlockSpec((_ROWS, _CB), lambda i, j: (j * _NRB + i, 0)),
        out_shape=jax.ShapeDtypeStruct((_NCB * _N, _CB), jnp.float32),
    )(parts, h, w1)


def _mm2_body(parts_ref, raw_ref, h_ref, w_ref, b_ref, o_ref):
    dinv = _dinv_from(parts_ref[...])
    z = (dinv[:, None] * raw_ref[...]
         + jnp.dot(h_ref[...], w_ref[...], preferred_element_type=jnp.float32)
         + b_ref[...][None, :])
    o_ref[...] = jnp.maximum(z, 0.0)


def _mm2(parts, raw4, h, w2, b):
    in_dim = h.shape[1]
    return pl.pallas_call(
        _mm2_body,
        grid=(_NRB, _NCB),
        in_specs=[
            pl.BlockSpec((_NSC, _ROWS, _CB), lambda i, j: (0, i, 0)),
            pl.BlockSpec((_ROWS, _CB), lambda i, j: (j * _NRB + i, 0)),
            pl.BlockSpec((_ROWS, in_dim), lambda i, j: (i, 0)),
            pl.BlockSpec((in_dim, _CB), lambda i, j: (0, j)),
            pl.BlockSpec((_CB,), lambda i, j: (j,)),
        ],
        out_specs=pl.BlockSpec((_ROWS, _CB), lambda i, j: (i, j)),
        out_shape=jax.ShapeDtypeStruct((_N, _NCB * _CB), jnp.float32),
    )(parts, raw4, h, w2, b)


def _dense_body(h_ref, w_ref, b_ref, o_ref):
    o_ref[...] = jnp.dot(
        h_ref[...], w_ref[...],
        preferred_element_type=jnp.float32) + b_ref[...][None, :]


def _dense(h, dw, db):
    k, o = dw.shape
    return pl.pallas_call(
        _dense_body,
        grid=(_NRB,),
        in_specs=[
            pl.BlockSpec((_ROWS, k), lambda i: (i, 0)),
            pl.BlockSpec((k, o), lambda i: (0, 0)),
            pl.BlockSpec((o,), lambda i: (0,)),
        ],
        out_specs=pl.BlockSpec((_ROWS, o), lambda i: (i, 0)),
        out_shape=jax.ShapeDtypeStruct((_N, o), jnp.float32),
    )(h, dw, db)


def kernel(x, edge_index, layers, dense_w, dense_b):
    row = edge_index[0]
    col = edge_index[1]
    rowp = jnp.concatenate([row, jnp.full((_PAD,), _N, jnp.int32)])
    colp = jnp.concatenate([col, jnp.zeros((_PAD,), jnp.int32)])
    col4 = colp[None, :] + (jnp.arange(_NCB, dtype=jnp.int32) * _N)[:, None]
    parts = _deg_call(rowp)
    h = x
    for (w1, w2, b) in layers:
        y4 = _mm1(parts, h, w1)
        raw4 = _agg_call(y4, col4, rowp)
        h = _mm2(parts, raw4, h, w2, b)
    return _dense(h, dense_w, dense_b)
